# Initial kernel scaffold; baseline (speedup 1.0000x reference)
import jax
import jax.numpy as jnp
from jax.experimental import pallas as pl


def kernel(hidden_states, top_k_index, top_k_weights, w_gate, w_up, w_down):
    raise NotImplementedError("write your pallas kernel here")


# speedup vs baseline: 2.0842x; 2.0842x over previous
import jax
import jax.numpy as jnp
from jax import lax
from jax.experimental import pallas as pl
from jax.experimental.pallas import tpu as pltpu

NUM_EXPERTS = 8
TOP_K = 2
HIDDEN = 1024
INTER = 768
SEQ = 2048

TB = 512


def _moe_body(x_ref, coef_ref, wg_ref, wu_ref, wd_ref, out_ref):
    e = pl.program_id(1)
    x = x_ref[...]
    wg = wg_ref[0]
    wu = wu_ref[0]
    wd = wd_ref[0]
    g = lax.dot_general(x, wg, (((1,), (1,)), ((), ())),
                        preferred_element_type=jnp.float32)
    u = lax.dot_general(x, wu, (((1,), (1,)), ((), ())),
                        preferred_element_type=jnp.float32)
    h = g * lax.logistic(g) * u
    y = lax.dot_general(h, wd, (((1,), (1,)), ((), ())),
                        preferred_element_type=jnp.float32)
    lane = lax.broadcasted_iota(jnp.int32, (TB, NUM_EXPERTS), 1)
    coef_col = jnp.sum(jnp.where(lane == e, coef_ref[...], 0.0), axis=1,
                       keepdims=True)
    y = y * coef_col

    @pl.when(e == 0)
    def _():
        out_ref[...] = y

    @pl.when(e != 0)
    def _():
        out_ref[...] += y


def kernel(hidden_states, top_k_index, top_k_weights, w_gate, w_up, w_down):
    orig_shape = hidden_states.shape
    x = hidden_states.reshape(-1, HIDDEN)
    ids = top_k_index.reshape(-1, TOP_K)
    tw = top_k_weights.reshape(-1, TOP_K)
    onehot = (ids[..., None] == jnp.arange(NUM_EXPERTS, dtype=jnp.int32)).astype(jnp.float32)
    coef = jnp.einsum("tk,tke->te", tw, onehot)

    grid = (SEQ // TB, NUM_EXPERTS)
    out = pl.pallas_call(
        _moe_body,
        grid=grid,
        in_specs=[
            pl.BlockSpec((TB, HIDDEN), lambda t, e: (t, 0)),
            pl.BlockSpec((TB, NUM_EXPERTS), lambda t, e: (t, 0)),
            pl.BlockSpec((1, INTER, HIDDEN), lambda t, e: (e, 0, 0)),
            pl.BlockSpec((1, INTER, HIDDEN), lambda t, e: (e, 0, 0)),
            pl.BlockSpec((1, HIDDEN, INTER), lambda t, e: (e, 0, 0)),
        ],
        out_specs=pl.BlockSpec((TB, HIDDEN), lambda t, e: (t, 0)),
        out_shape=jax.ShapeDtypeStruct((SEQ, HIDDEN), jnp.float32),
        compiler_params=pltpu.CompilerParams(
            dimension_semantics=("parallel", "arbitrary"),
        ),
    )(x, coef, w_gate, w_up, w_down)
    return out.reshape(orig_shape)
